# baseline (speedup 1.0000x reference)
import functools

import jax
import jax.numpy as jnp
from jax import lax
from jax.experimental import pallas as pl
from jax.experimental.pallas import tpu as pltpu
from jax.experimental.pallas import tpu_sc as plsc

STATE_NUM = 100000
B = 16384
D = 128
N = 4
ROW = D * N
L = 16
NC, NS = 2, 16
NW = NC * NS
BPW = B // NW
C = 16
NCH = BPW // C
RING = 6
JV = ROW // L


def _body(e1_hbm, e2_hbm, xidx_hbm, yidx_hbm, out_hbm,
          xidx_v, yidx_v, rbuf1, rbuf2, accs_v, out_v, sems1, sems2):
  wid = lax.axis_index("s") * NC + lax.axis_index("c")
  base = wid * BPW

  pltpu.sync_copy(xidx_hbm.at[pl.ds(base, BPW)], xidx_v)
  pltpu.sync_copy(yidx_hbm.at[pl.ds(base, BPW)], yidx_v)

  def issue(g, slot):
    pltpu.make_async_copy(e1_hbm.at[xidx_v.at[pl.ds(g * C, C)]],
                          rbuf1.at[slot], sems1.at[slot]).start()
    pltpu.make_async_copy(e2_hbm.at[yidx_v.at[pl.ds(g * C, C)]],
                          rbuf2.at[slot], sems2.at[slot]).start()

  def wait_chunk(g, slot):
    pltpu.make_async_copy(e1_hbm.at[xidx_v.at[pl.ds(g * C, C)]],
                          rbuf1.at[slot], sems1.at[slot]).wait()
    pltpu.make_async_copy(e2_hbm.at[yidx_v.at[pl.ds(g * C, C)]],
                          rbuf2.at[slot], sems2.at[slot]).wait()

  for s in range(RING - 1):
    issue(s, s)

  def g_body(g, slot):
    wait_chunk(g, slot)

    def row_body(r, c2):
      acc = rbuf1[slot, r, pl.ds(0, L)] * rbuf2[slot, r, pl.ds(0, L)]
      for j in range(1, JV):
        acc = acc + (rbuf1[slot, r, pl.ds(j * L, L)]
                     * rbuf2[slot, r, pl.ds(j * L, L)])
      accs_v[pl.ds((g * C + r) * L, L)] = acc
      return c2

    lax.fori_loop(0, C, row_body, 0)

    s5 = slot - 1
    s5 = lax.select(s5 < 0, s5 + RING, s5)

    @pl.when(g + RING - 1 < NCH)
    def _():
      issue(g + RING - 1, s5)

    nxt = slot + 1
    return lax.select(nxt >= RING, nxt - RING, nxt)

  lax.fori_loop(0, NCH, g_body, 0)

  iota = lax.iota(jnp.int32, L)
  basev = lax.shift_left(lax.shift_right_logical(iota, 2), 4) + (iota & 3)

  def fold_body(t, carry):
    idx0 = basev + t * (N * L)
    v = plsc.load_gather(accs_v, [idx0])
    v = v + plsc.load_gather(accs_v, [idx0 + N])
    v = v + plsc.load_gather(accs_v, [idx0 + 2 * N])
    v = v + plsc.load_gather(accs_v, [idx0 + 3 * N])
    out_v[pl.ds(t * L, L)] = v
    return carry

  lax.fori_loop(0, BPW * N // L, fold_body, 0)

  pltpu.sync_copy(out_v, out_hbm.at[pl.ds(wid * BPW * N, BPW * N)])


_mesh = plsc.VectorSubcoreMesh(core_axis_name="c", subcore_axis_name="s",
                               num_cores=NC, num_subcores=NS)

_call = pl.kernel(
    _body,
    out_type=jax.ShapeDtypeStruct((B * N,), jnp.float32),
    mesh=_mesh,
    compiler_params=pltpu.CompilerParams(needs_layout_passes=False, skip_device_barrier=True),
    scratch_types=[
        pltpu.VMEM((BPW,), jnp.int32),
        pltpu.VMEM((BPW,), jnp.int32),
        pltpu.VMEM((RING, C, ROW), jnp.float32),
        pltpu.VMEM((RING, C, ROW), jnp.float32),
        pltpu.VMEM((BPW * L,), jnp.float32),
        pltpu.VMEM((BPW * N,), jnp.float32),
        pltpu.SemaphoreType.DMA((RING,)),
        pltpu.SemaphoreType.DMA((RING,)),
    ],
)


@jax.jit
def kernel(state, embedding_1, embedding_2):
  x = state[:, 0]
  y = state[:, 1]
  e1 = embedding_1.reshape(STATE_NUM, ROW)
  e2 = embedding_2.reshape(STATE_NUM, ROW)
  out = _call(e1, e2, x, y)
  return out.reshape(B, N)


# speedup vs baseline: 1.0010x; 1.0010x over previous
import jax
import jax.numpy as jnp
from jax import lax
from jax.experimental import pallas as pl
from jax.experimental.pallas import tpu as pltpu
from jax.experimental.pallas import tpu_sc as plsc

STATE_NUM = 100000
B = 16384
D = 128
N = 4
ROW = D * N
L = 16
NC, NS = 2, 16
NW = NC * NS
BPW = B // NW
C = 16
NCH = BPW // C
RING = 6
JV = ROW // L


def _body(e1_hbm, e2_hbm, xidx_hbm, yidx_hbm, out_hbm,
          xidx_v, yidx_v, rbuf1, rbuf2, accs_v, out_v, sems1, sems2):
  wid = lax.axis_index("s") * NC + lax.axis_index("c")
  base = wid * BPW

  pltpu.sync_copy(xidx_hbm.at[pl.ds(base, BPW)], xidx_v)
  pltpu.sync_copy(yidx_hbm.at[pl.ds(base, BPW)], yidx_v)

  def issue(g, slot):
    pltpu.make_async_copy(e1_hbm.at[xidx_v.at[pl.ds(g * C, C)]],
                          rbuf1.at[slot], sems1.at[slot]).start()
    pltpu.make_async_copy(e2_hbm.at[yidx_v.at[pl.ds(g * C, C)]],
                          rbuf2.at[slot], sems2.at[slot]).start()

  def wait_chunk(g, slot):
    pltpu.make_async_copy(e1_hbm.at[xidx_v.at[pl.ds(g * C, C)]],
                          rbuf1.at[slot], sems1.at[slot]).wait()
    pltpu.make_async_copy(e2_hbm.at[yidx_v.at[pl.ds(g * C, C)]],
                          rbuf2.at[slot], sems2.at[slot]).wait()

  for s in range(RING - 1):
    issue(s, s)

  def g_body(g, slot):
    wait_chunk(g, slot)

    def row_body(r, c2):
      acc = rbuf1[slot, r, pl.ds(0, L)] * rbuf2[slot, r, pl.ds(0, L)]
      for j in range(1, JV):
        acc = acc + (rbuf1[slot, r, pl.ds(j * L, L)]
                     * rbuf2[slot, r, pl.ds(j * L, L)])
      accs_v[pl.ds((g * C + r) * L, L)] = acc
      return c2

    lax.fori_loop(0, C, row_body, 0)

    s5 = slot - 1
    s5 = lax.select(s5 < 0, s5 + RING, s5)

    @pl.when(g + RING - 1 < NCH)
    def _():
      issue(g + RING - 1, s5)

    nxt = slot + 1
    return lax.select(nxt >= RING, nxt - RING, nxt)

  lax.fori_loop(0, NCH, g_body, 0)

  iota = lax.iota(jnp.int32, L)
  basev = lax.shift_left(lax.shift_right_logical(iota, 2), 4) + (iota & 3)

  def fold_body(t, carry):
    idx0 = basev + t * (N * L)
    v = plsc.load_gather(accs_v, [idx0])
    v = v + plsc.load_gather(accs_v, [idx0 + N])
    v = v + plsc.load_gather(accs_v, [idx0 + 2 * N])
    v = v + plsc.load_gather(accs_v, [idx0 + 3 * N])
    out_v[pl.ds(t * L, L)] = v
    return carry

  lax.fori_loop(0, BPW * N // L, fold_body, 0)

  pltpu.sync_copy(out_v, out_hbm.at[pl.ds(wid * BPW * N, BPW * N)])


_mesh = plsc.VectorSubcoreMesh(core_axis_name="c", subcore_axis_name="s",
                               num_cores=NC, num_subcores=NS)

_call = pl.kernel(
    _body,
    out_type=jax.ShapeDtypeStruct((B * N,), jnp.float32),
    mesh=_mesh,
    compiler_params=pltpu.CompilerParams(needs_layout_passes=False),
    scratch_types=[
        pltpu.VMEM((BPW,), jnp.int32),
        pltpu.VMEM((BPW,), jnp.int32),
        pltpu.VMEM((RING, C, ROW), jnp.float32),
        pltpu.VMEM((RING, C, ROW), jnp.float32),
        pltpu.VMEM((BPW * L,), jnp.float32),
        pltpu.VMEM((BPW * N,), jnp.float32),
        pltpu.SemaphoreType.DMA((RING,)),
        pltpu.SemaphoreType.DMA((RING,)),
    ],
)


@jax.jit
def kernel(state, embedding_1, embedding_2):
  x = state[:, 0]
  y = state[:, 1]
  e1 = embedding_1.reshape(STATE_NUM, ROW)
  e2 = embedding_2.reshape(STATE_NUM, ROW)
  out = _call(e1, e2, x, y)
  return out.reshape(B, N)
